# baseline (speedup 1.0000x reference)
import functools

import jax
import jax.numpy as jnp
from jax.experimental import pallas as pl

N = 50000
E = 800000
F = 64
G = 64

_ET = 3200


def _edge_mlp_body(xsrc_ref, deltaT_ref, w1a_ref, w1b_ref, b1_ref, w2_ref,
                   b2_ref, out_ref):
    x = xsrc_ref[...]
    dT = deltaT_ref[...]
    m1 = jax.lax.dot(x, w1a_ref[...], preferred_element_type=jnp.float32)
    m1 = m1 + jax.lax.dot_general(dT, w1b_ref[...], (((0,), (0,)), ((), ())),
                                  preferred_element_type=jnp.float32)
    m1 = jnp.maximum(m1 + b1_ref[...], 0.0)
    out_ref[...] = jax.lax.dot(m1, w2_ref[...],
                               preferred_element_type=jnp.float32) + b2_ref[...]


def _edge_mlp(xsrc, deltaT, w1a, w1b, b1, w2, b2):
    grid = (E // _ET,)
    return pl.pallas_call(
        _edge_mlp_body,
        grid=grid,
        in_specs=[
            pl.BlockSpec((_ET, F), lambda i: (i, 0)),
            pl.BlockSpec((8, _ET), lambda i: (0, i)),
            pl.BlockSpec((F, F), lambda i: (0, 0)),
            pl.BlockSpec((8, F), lambda i: (0, 0)),
            pl.BlockSpec((1, F), lambda i: (0, 0)),
            pl.BlockSpec((F, F), lambda i: (0, 0)),
            pl.BlockSpec((1, F), lambda i: (0, 0)),
        ],
        out_specs=pl.BlockSpec((_ET, F), lambda i: (i, 0)),
        out_shape=jax.ShapeDtypeStruct((E, F), jnp.float32),
    )(xsrc, deltaT, w1a, w1b, b1, w2, b2)


def kernel(pos, edge_index, batch, c1_w1, c1_b1, c1_w2, c1_b2, c2_w1, c2_b1,
           c2_w2, c2_b2, r1_w, r1_b, r2_w, r2_b):
    mask = jnp.zeros((N,), dtype=bool).at[edge_index.reshape(-1)].set(True)
    assoc = jnp.cumsum(mask.astype(jnp.int32)) - 1
    ei = assoc[edge_index]
    src, dst = ei[0], ei[1]

    delta = pos[src] - pos[dst]
    deltaT = jnp.zeros((8, E), jnp.float32).at[:3, :].set(delta.T)

    def layer(h, w1, b1, w2, b2):
        w1a = w1[:F]
        w1b = jnp.zeros((8, F), jnp.float32).at[:3, :].set(w1[F:])
        xsrc = h[src]
        m = _edge_mlp(xsrc, deltaT, w1a, w1b, b1[None, :], w2, b2[None, :])
        out = jax.ops.segment_max(m, dst, num_segments=N)
        return jnp.maximum(out, 0.0)

    w1a_1 = jnp.zeros((F, F), jnp.float32).at[:3, :].set(c1_w1[:3])
    w1b_1 = jnp.zeros((8, F), jnp.float32).at[:3, :].set(c1_w1[3:])
    xsrc1 = jnp.zeros((N, F), jnp.float32).at[:, :3].set(pos)[src]
    m = _edge_mlp(xsrc1, deltaT, w1a_1, w1b_1, c1_b1[None, :], c1_w2,
                  c1_b2[None, :])
    h = jnp.maximum(jax.ops.segment_max(m, dst, num_segments=N), 0.0)

    h = layer(h, c2_w1, c2_b1, c2_w2, c2_b2)
    h = layer(h, c2_w1, c2_b1, c2_w2, c2_b2)

    sums = jax.ops.segment_sum(h, batch, num_segments=G)
    cnt = jax.ops.segment_sum(jnp.ones((N,), jnp.float32), batch,
                              num_segments=G)
    mean = sums / jnp.maximum(cnt, 1.0)[:, None]
    return (mean @ r1_w + r1_b, mean @ r2_w + r2_b)
